# baseline (speedup 1.0000x reference)
import jax
import jax.numpy as jnp
from jax import lax
from jax.experimental import pallas as pl
from jax.experimental.pallas import tpu as pltpu
from jax.experimental.pallas import tpu_sc as plsc

_D = 128
_G = 64
_C = 10
_NPAD = 10240
_QH = 2560
_NSL = 80
_CAP = 160
_GRP = 16
_NBUF = 4
_PADDST = 1 << 20


def _sc_route(lo):
  mesh = plsc.VectorSubcoreMesh(core_axis_name="c", subcore_axis_name="s")
  i32 = jnp.int32
  outs = [jax.ShapeDtypeStruct((32, _NSL, 128), i32) for _ in range(4)]
  outs.append(jax.ShapeDtypeStruct((512,), i32))
  scratch = (
      [pltpu.VMEM((_NSL, 128), i32) for _ in range(2)]
      + [pltpu.VMEM((_NSL, 128), i32) for _ in range(4)]
      + [pltpu.VMEM((16,), i32)]
  )

  def body(src_hbm, dst_hbm, cpad_hbm,
           oas, oad, obs, obd, cnts,
           in_s, in_d, bas, bad, bbs, bbd, cst):
    cid = lax.axis_index("c")
    sid = lax.axis_index("s")
    w = cid * 16 + sid
    pltpu.sync_copy(src_hbm.at[w], in_s)
    pltpu.sync_copy(dst_hbm.at[w], in_d)
    pltpu.sync_copy(cpad_hbm, bas)
    pltpu.sync_copy(cpad_hbm, bad)
    pltpu.sync_copy(cpad_hbm, bbs)
    pltpu.sync_copy(cpad_hbm, bbd)

    lane = lax.broadcasted_iota(i32, (16,), 0)

    def chunk(cc, offs):
      off_a, off_b = offs
      for v in range(8):
        s16 = in_s[cc, pl.ds(v * 16, 16)]
        d16 = in_d[cc, pl.ds(v * 16, 16)]
        m_a = jnp.logical_and(d16 >= lo, d16 < lo + _QH)
        m_b = jnp.logical_and(d16 >= lo + _QH, d16 < lo + 2 * _QH)
        pk_a = (s16 << 12) | jnp.where(m_a, d16 - lo, _QH)
        pk_b = (s16 << 12) | jnp.where(m_b, d16 - lo - _QH, _QH)
        ka = jnp.where(m_a, 0, 1).astype(i32)
        kb = jnp.where(m_b, 0, 1).astype(i32)
        _, va = lax.sort((ka, pk_a), num_keys=1)
        _, vb = lax.sort((kb, pk_b), num_keys=1)
        na = plsc.all_reduce_population_count(m_a)
        nb = plsc.all_reduce_population_count(m_b)
        pos_a = off_a + lane
        pos_b = off_b + lane
        wa = lane < na
        wb = lane < nb
        plsc.store_scatter(bas, [lax.shift_right_logical(pos_a, 7),
                                 jnp.bitwise_and(pos_a, 127)],
                           lax.shift_right_logical(va, 12), mask=wa)
        plsc.store_scatter(bad, [lax.shift_right_logical(pos_a, 7),
                                 jnp.bitwise_and(pos_a, 127)],
                           jnp.bitwise_and(va, 4095), mask=wa)
        plsc.store_scatter(bbs, [lax.shift_right_logical(pos_b, 7),
                                 jnp.bitwise_and(pos_b, 127)],
                           lax.shift_right_logical(vb, 12), mask=wb)
        plsc.store_scatter(bbd, [lax.shift_right_logical(pos_b, 7),
                                 jnp.bitwise_and(pos_b, 127)],
                           jnp.bitwise_and(vb, 4095), mask=wb)
        off_a = off_a + na
        off_b = off_b + nb
      return (off_a, off_b)

    zeros16 = jnp.zeros((16,), i32)
    off_a, off_b = lax.fori_loop(0, _NSL, chunk, (zeros16, zeros16))
    cst[...] = jnp.where(lane == 0, off_a,
                         jnp.where(lane == 1, off_b, 0))
    pltpu.sync_copy(bas, oas.at[w])
    pltpu.sync_copy(bad, oad.at[w])
    pltpu.sync_copy(bbs, obs.at[w])
    pltpu.sync_copy(bbd, obd.at[w])
    pltpu.sync_copy(cst, cnts.at[pl.ds(w * 16, 16)])

  return pl.kernel(
      body, out_type=outs, mesh=mesh, scratch_types=scratch,
      compiler_params=pltpu.CompilerParams(needs_layout_passes=False))


def _sc_qsum(with_deg):
  mesh = plsc.VectorSubcoreMesh(core_axis_name="c", subcore_axis_name="s")
  rows_per_tile = _QH // 16
  acc_rows = _QH + 128
  outs = [jax.ShapeDtypeStruct((2 * _QH, _D), jnp.float32)]
  if with_deg:
    outs.append(jax.ShapeDtypeStruct((2 * _QH,), jnp.float32))
  scratch = (
      [pltpu.VMEM((_GRP, 64), jnp.int32) for _ in range(2)]
      + [pltpu.VMEM((64, _D), jnp.float32) for _ in range(_NBUF)]
      + [pltpu.VMEM((64,), jnp.float32)]
      + [pltpu.VMEM((rows_per_tile,), jnp.float32)]
      + [pltpu.VMEM((512,), jnp.int32)]
      + [pltpu.VMEM_SHARED((acc_rows, _D), jnp.float32),
         pltpu.VMEM_SHARED((acc_rows,), jnp.float32)]
      + [pltpu.SemaphoreType.DMA for _ in range(2 * _NBUF + 1)]
  )

  def body(tbl_hbm, las, lad, lbs, lbd, cnts_hbm, z2_hbm, z1_hbm, *rest):
    if with_deg:
      acc_out, deg_out = rest[0], rest[1]
      rest = rest[2:]
    else:
      acc_out, deg_out = rest[0], None
      rest = rest[1:]
    win_s, win_d = rest[0], rest[1]
    bufs = list(rest[2:2 + _NBUF])
    ones_v = rest[2 + _NBUF]
    deg_stage = rest[3 + _NBUF]
    cnt_v = rest[4 + _NBUF]
    acc_sh = rest[5 + _NBUF]
    deg_sh = rest[6 + _NBUF]
    gsems = list(rest[7 + _NBUF:7 + 2 * _NBUF])
    ssems = list(rest[7 + 2 * _NBUF:7 + 3 * _NBUF])
    dsem = rest[7 + 3 * _NBUF]

    cid = lax.axis_index("c")
    sid = lax.axis_index("s")
    r0 = sid * rows_per_tile

    pltpu.sync_copy(z2_hbm.at[pl.ds(r0, rows_per_tile)],
                    acc_sh.at[pl.ds(r0, rows_per_tile)])
    if with_deg:
      pltpu.sync_copy(z1_hbm.at[pl.ds(r0, rows_per_tile)], deg_stage)
      pltpu.sync_copy(deg_stage, deg_sh.at[pl.ds(r0, rows_per_tile)])
      for i in range(64 // 16):
        ones_v[pl.ds(i * 16, 16)] = jnp.full((16,), 1.0, jnp.float32)
    pltpu.sync_copy(cnts_hbm, cnt_v)
    plsc.subcore_barrier()

    def wait_gather(b):
      pltpu.make_async_copy(
          tbl_hbm.at[win_s.at[0]], bufs[b], gsems[b]).wait()

    def wait_scatter(b):
      pltpu.make_async_copy(
          bufs[b], acc_sh.at[win_d.at[0]], ssems[b]).wait()

    def run_region(lsrc, ldst, w):
      cvec = cnt_v[pl.ds(pl.multiple_of(w * 16, 16), 16)]
      lane16 = lax.broadcasted_iota(jnp.int32, (16,), 0)
      cnt = jnp.sum(jnp.where(lane16 == cid, cvec, 0))

      def group(gi, carry):
        @pl.when(gi * (_GRP * 64) < cnt)
        def _():
          g0 = pl.multiple_of(gi * _GRP, 8)
          pltpu.sync_copy(lsrc.at[w, pl.ds(g0, _GRP)], win_s)
          pltpu.sync_copy(ldst.at[w, pl.ds(g0, _GRP)], win_d)
          for b in range(_NBUF - 1):
            pltpu.async_copy(tbl_hbm.at[win_s.at[b]], bufs[b], gsems[b])
          for c in range(_GRP):
            b = c % _NBUF
            wait_gather(b)
            pltpu.async_copy(bufs[b], acc_sh.at[win_d.at[c]], ssems[b],
                             add=True)
            if with_deg:
              pltpu.async_copy(ones_v, deg_sh.at[win_d.at[c]], dsem,
                               add=True)
            if c + _NBUF - 1 < _GRP:
              nb = (c + _NBUF - 1) % _NBUF
              if c >= 1:
                wait_scatter(nb)
              pltpu.async_copy(tbl_hbm.at[win_s.at[c + _NBUF - 1]],
                               bufs[nb], gsems[nb])
          for c in range(_GRP - _NBUF, _GRP):
            wait_scatter(c % _NBUF)
          if with_deg:
            for _ in range(_GRP):
              pltpu.make_async_copy(
                  ones_v, deg_sh.at[win_d.at[0]], dsem).wait()
        return carry

      lax.fori_loop(0, _CAP // _GRP, group, 0)

    @pl.when(cid == 0)
    def _():
      run_region(las, lad, 2 * sid)
      run_region(las, lad, 2 * sid + 1)

    @pl.when(cid == 1)
    def _():
      run_region(lbs, lbd, 2 * sid)
      run_region(lbs, lbd, 2 * sid + 1)

    plsc.subcore_barrier()

    g0 = pl.multiple_of(cid * _QH + r0, 8)
    pltpu.sync_copy(acc_sh.at[pl.ds(r0, rows_per_tile)],
                    acc_out.at[pl.ds(g0, rows_per_tile)])
    if with_deg:
      pltpu.sync_copy(deg_sh.at[pl.ds(r0, rows_per_tile)], deg_stage)
      pltpu.sync_copy(deg_stage, deg_out.at[pl.ds(g0, rows_per_tile)])

  return pl.kernel(
      body, out_type=outs, mesh=mesh, scratch_types=scratch,
      compiler_params=pltpu.CompilerParams(needs_layout_passes=False))


def _dot_t(a, b):
  return lax.dot_general(a, b, (((1,), (1,)), ((), ())),
                         preferred_element_type=jnp.float32)


def _tc_layer1(nblk, n_pad):
  def body(p0, d0, xb, bb, wrel, wroot, bias, h_out, s1_out, s1_acc):
    i = pl.program_id(0)
    deg = jnp.maximum(d0[...], 1.0)
    m = p0[...] / deg
    h = _dot_t(m, wrel[...]) + _dot_t(xb[...], wroot[...]) + bias[...]
    h = jnp.maximum(h, 0.0)
    h_out[...] = h
    g = lax.broadcasted_iota(jnp.int32, (_G, 128), 0)
    onehot = (bb[0] == g).astype(jnp.float32)
    ps = lax.dot_general(onehot, h, (((1,), (0,)), ((), ())),
                         preferred_element_type=jnp.float32)

    @pl.when(i == 0)
    def _():
      s1_acc[...] = jnp.zeros_like(s1_acc)

    s1_acc[...] += ps

    @pl.when(i == nblk - 1)
    def _():
      s1_out[...] = s1_acc[...]

  bs2 = lambda: pl.BlockSpec((128, 128), lambda i: (i, 0))
  col = lambda: pl.BlockSpec((128, 1), lambda i: (i, 0))
  full = lambda r, c: pl.BlockSpec((r, c), lambda i: (0, 0))
  return pl.pallas_call(
      body,
      grid=(nblk,),
      in_specs=[bs2(), col(), bs2(),
                pl.BlockSpec((1, 1, 128), lambda i: (i, 0, 0)),
                full(128, 128), full(128, 128), full(1, 128)],
      out_specs=[bs2(), pl.BlockSpec((_G, 128), lambda i: (0, 0))],
      out_shape=[jax.ShapeDtypeStruct((n_pad, 128), jnp.float32),
                 jax.ShapeDtypeStruct((_G, 128), jnp.float32)],
      scratch_shapes=[pltpu.VMEM((_G, 128), jnp.float32)],
      compiler_params=pltpu.CompilerParams(
          dimension_semantics=("arbitrary",)),
  )


def _tc_layer2_head(nblk, n_pad):
  def body(q0, d0, hb, bb, wrel, wroot, bias, s1,
           wa, wb, bl1, w2, bl2, out, ps_acc, cnt_acc):
    i = pl.program_id(0)
    deg = jnp.maximum(d0[...], 1.0)
    m = q0[...] / deg
    h = _dot_t(m, wrel[...]) + _dot_t(hb[...], wroot[...]) + bias[...]
    h = jnp.maximum(h, 0.0)
    g = lax.broadcasted_iota(jnp.int32, (_G, 128), 0)
    onehot = (bb[0] == g).astype(jnp.float32)
    ps = lax.dot_general(onehot, h, (((1,), (0,)), ((), ())),
                         preferred_element_type=jnp.float32)
    cnt = jnp.sum(onehot, axis=1, keepdims=True)

    @pl.when(i == 0)
    def _():
      ps_acc[...] = jnp.zeros_like(ps_acc)
      cnt_acc[...] = jnp.zeros_like(cnt_acc)

    ps_acc[...] += ps
    cnt_acc[...] += jnp.broadcast_to(cnt, (_G, 128))

    @pl.when(i == nblk - 1)
    def _():
      c = jnp.maximum(cnt_acc[...], 1.0)
      pool1 = s1[...] / c
      pool2 = ps_acc[...] / c
      hid = jnp.maximum(_dot_t(pool1, wa[...]) + _dot_t(pool2, wb[...])
                        + bl1[...], 0.0)
      logits = _dot_t(hid, w2[...]) + bl2[...]
      valid = lax.broadcasted_iota(jnp.int32, (_G, 128), 1) < _C
      lm = jnp.where(valid, logits, -1e30)
      mx = jnp.max(lm, axis=1, keepdims=True)
      lse = jnp.log(jnp.sum(jnp.exp(lm - mx), axis=1, keepdims=True))
      out[...] = lm - mx - lse

  bs2 = lambda: pl.BlockSpec((128, 128), lambda i: (i, 0))
  col = lambda: pl.BlockSpec((128, 1), lambda i: (i, 0))
  full = lambda r, c: pl.BlockSpec((r, c), lambda i: (0, 0))
  return pl.pallas_call(
      body,
      grid=(nblk,),
      in_specs=[bs2(), col(), bs2(),
                pl.BlockSpec((1, 1, 128), lambda i: (i, 0, 0)),
                full(128, 128), full(128, 128), full(1, 128),
                full(_G, 128),
                full(128, 128), full(128, 128), full(1, 128),
                full(128, 128), full(1, 128)],
      out_specs=[pl.BlockSpec((_G, 128), lambda i: (0, 0))],
      out_shape=[jax.ShapeDtypeStruct((_G, 128), jnp.float32)],
      scratch_shapes=[pltpu.VMEM((_G, 128), jnp.float32),
                      pltpu.VMEM((_G, 128), jnp.float32)],
      compiler_params=pltpu.CompilerParams(
          dimension_semantics=("arbitrary",)),
  )


def kernel(x, edge_index, batch, W_rel1, b1, W_root1, W_rel2, b2, W_root2,
           W_lin1, b_lin1, W_lin2, b_lin2):
  n, d = x.shape
  e = edge_index.shape[1]
  n_pad = _NPAD
  nblk = n_pad // 128
  e_pad = 32 * _NSL * 128

  src = edge_index[0]
  dst = edge_index[1]
  srcp = jnp.concatenate(
      [src, jnp.zeros((e_pad - e,), jnp.int32)]).reshape(32, _NSL, 128)
  dstp = jnp.concatenate(
      [dst, jnp.full((e_pad - e,), _PADDST, jnp.int32)]
  ).reshape(32, _NSL, 128)
  cpad = jnp.full((_NSL, 128), _QH, jnp.int32)
  xp = jnp.zeros((n_pad, d), jnp.float32).at[:n].set(x)
  batchp = jnp.concatenate(
      [batch, jnp.full((n_pad - n,), _G, jnp.int32)]).reshape(nblk, 1, 128)
  z2 = jnp.zeros((n_pad, d), jnp.float32)
  z1 = jnp.zeros((n_pad,), jnp.float32)

  l0s, l0d, l1s, l1d, c01 = _sc_route(0)(srcp, dstp, cpad)
  l2s, l2d, l3s, l3d, c23 = _sc_route(2 * _QH)(srcp, dstp, cpad)
  rs = lambda a: a.reshape(32, _CAP, 64)

  qsum_deg = _sc_qsum(True)
  alo, dlo = qsum_deg(xp, rs(l0s), rs(l0d), rs(l1s), rs(l1d), c01, z2, z1)
  ahi, dhi = qsum_deg(xp, rs(l2s), rs(l2d), rs(l3s), rs(l3d), c23, z2, z1)
  acc1 = jnp.concatenate([alo, ahi], axis=0)
  d0 = jnp.concatenate([dlo, dhi]).reshape(n_pad, 1)
  h1, s1 = _tc_layer1(nblk, n_pad)(
      acc1, d0, xp, batchp, W_rel1, W_root1, b1.reshape(1, 128))

  qsum = _sc_qsum(False)
  a2lo = qsum(h1, rs(l0s), rs(l0d), rs(l1s), rs(l1d), c01, z2, z1)
  a2hi = qsum(h1, rs(l2s), rs(l2d), rs(l3s), rs(l3d), c23, z2, z1)
  if isinstance(a2lo, (tuple, list)):
    a2lo = a2lo[0]
  if isinstance(a2hi, (tuple, list)):
    a2hi = a2hi[0]
  acc2 = jnp.concatenate([a2lo, a2hi], axis=0)

  w2p = jnp.zeros((128, 128), jnp.float32).at[:_C].set(W_lin2)
  bl2p = jnp.zeros((1, 128), jnp.float32).at[0, :_C].set(b_lin2)
  (logits,) = _tc_layer2_head(nblk, n_pad)(
      acc2, d0, h1, batchp,
      W_rel2, W_root2, b2.reshape(1, 128), s1,
      W_lin1[:, :128], W_lin1[:, 128:], b_lin1.reshape(1, 128),
      w2p, bl2p)
  return logits[:, :_C]


# speedup vs baseline: 1.0007x; 1.0007x over previous
import jax
import jax.numpy as jnp
from jax import lax
from jax.experimental import pallas as pl
from jax.experimental.pallas import tpu as pltpu
from jax.experimental.pallas import tpu_sc as plsc

_D = 128
_G = 64
_C = 10
_NPAD = 10240
_QH = 2560
_NSL = 80
_CAP = 160
_GRP = 16
_NBUF = 4
_PADDST = 1 << 20


def _sc_route(lo):
  mesh = plsc.VectorSubcoreMesh(core_axis_name="c", subcore_axis_name="s")
  i32 = jnp.int32
  outs = [jax.ShapeDtypeStruct((32, _NSL, 128), i32) for _ in range(4)]
  outs.append(jax.ShapeDtypeStruct((512,), i32))
  scratch = (
      [pltpu.VMEM((_NSL, 128), i32) for _ in range(2)]
      + [pltpu.VMEM((_NSL, 128), i32) for _ in range(4)]
      + [pltpu.VMEM((16,), i32)]
  )

  def body(src_hbm, dst_hbm, cpad_hbm,
           oas, oad, obs, obd, cnts,
           in_s, in_d, bas, bad, bbs, bbd, cst):
    cid = lax.axis_index("c")
    sid = lax.axis_index("s")
    w = cid * 16 + sid
    pltpu.sync_copy(src_hbm.at[w], in_s)
    pltpu.sync_copy(dst_hbm.at[w], in_d)
    pltpu.sync_copy(cpad_hbm, bas)
    pltpu.sync_copy(cpad_hbm, bad)
    pltpu.sync_copy(cpad_hbm, bbs)
    pltpu.sync_copy(cpad_hbm, bbd)

    lane = lax.broadcasted_iota(i32, (16,), 0)

    def chunk(cc, offs):
      off_a, off_b = offs
      for v in range(8):
        s16 = in_s[cc, pl.ds(v * 16, 16)]
        d16 = in_d[cc, pl.ds(v * 16, 16)]
        m_a = jnp.logical_and(d16 >= lo, d16 < lo + _QH)
        m_b = jnp.logical_and(d16 >= lo + _QH, d16 < lo + 2 * _QH)
        pk_a = (s16 << 12) | jnp.where(m_a, d16 - lo, _QH)
        pk_b = (s16 << 12) | jnp.where(m_b, d16 - lo - _QH, _QH)
        ka = jnp.where(m_a, 0, 1).astype(i32)
        kb = jnp.where(m_b, 0, 1).astype(i32)
        _, va = lax.sort((ka, pk_a), num_keys=1)
        _, vb = lax.sort((kb, pk_b), num_keys=1)
        na = plsc.all_reduce_population_count(m_a)
        nb = plsc.all_reduce_population_count(m_b)
        pos_a = off_a + lane
        pos_b = off_b + lane
        wa = lane < na
        wb = lane < nb
        plsc.store_scatter(bas, [lax.shift_right_logical(pos_a, 7),
                                 jnp.bitwise_and(pos_a, 127)],
                           lax.shift_right_logical(va, 12), mask=wa)
        plsc.store_scatter(bad, [lax.shift_right_logical(pos_a, 7),
                                 jnp.bitwise_and(pos_a, 127)],
                           jnp.bitwise_and(va, 4095), mask=wa)
        plsc.store_scatter(bbs, [lax.shift_right_logical(pos_b, 7),
                                 jnp.bitwise_and(pos_b, 127)],
                           lax.shift_right_logical(vb, 12), mask=wb)
        plsc.store_scatter(bbd, [lax.shift_right_logical(pos_b, 7),
                                 jnp.bitwise_and(pos_b, 127)],
                           jnp.bitwise_and(vb, 4095), mask=wb)
        off_a = off_a + na
        off_b = off_b + nb
      return (off_a, off_b)

    zeros16 = jnp.zeros((16,), i32)
    off_a, off_b = lax.fori_loop(0, _NSL, chunk, (zeros16, zeros16))
    cst[...] = jnp.where(lane == 0, off_a,
                         jnp.where(lane == 1, off_b, 0))
    pltpu.sync_copy(bas, oas.at[w])
    pltpu.sync_copy(bad, oad.at[w])
    pltpu.sync_copy(bbs, obs.at[w])
    pltpu.sync_copy(bbd, obd.at[w])
    pltpu.sync_copy(cst, cnts.at[pl.ds(w * 16, 16)])

  return pl.kernel(
      body, out_type=outs, mesh=mesh, scratch_types=scratch,
      compiler_params=pltpu.CompilerParams(needs_layout_passes=False))


def _sc_qsum(with_deg):
  mesh = plsc.VectorSubcoreMesh(core_axis_name="c", subcore_axis_name="s")
  rows_per_tile = _QH // 16
  acc_rows = _QH + 128
  outs = [jax.ShapeDtypeStruct((2 * _QH, _D), jnp.float32)]
  if with_deg:
    outs.append(jax.ShapeDtypeStruct((2 * _QH,), jnp.float32))
  scratch = (
      [pltpu.VMEM((_GRP, 64), jnp.int32) for _ in range(2)]
      + [pltpu.VMEM((64, _D), jnp.float32) for _ in range(_NBUF)]
      + [pltpu.VMEM((64,), jnp.float32)]
      + [pltpu.VMEM((rows_per_tile,), jnp.float32)]
      + [pltpu.VMEM((512,), jnp.int32)]
      + [pltpu.VMEM_SHARED((acc_rows, _D), jnp.float32),
         pltpu.VMEM_SHARED((acc_rows,), jnp.float32)]
      + [pltpu.SemaphoreType.DMA for _ in range(2 * _NBUF + 1)]
  )

  def body(tbl_hbm, las, lad, lbs, lbd, cnts_hbm, z2_hbm, z1_hbm, *rest):
    if with_deg:
      acc_out, deg_out = rest[0], rest[1]
      rest = rest[2:]
    else:
      acc_out, deg_out = rest[0], None
      rest = rest[1:]
    win_s, win_d = rest[0], rest[1]
    bufs = list(rest[2:2 + _NBUF])
    ones_v = rest[2 + _NBUF]
    deg_stage = rest[3 + _NBUF]
    cnt_v = rest[4 + _NBUF]
    acc_sh = rest[5 + _NBUF]
    deg_sh = rest[6 + _NBUF]
    gsems = list(rest[7 + _NBUF:7 + 2 * _NBUF])
    ssems = list(rest[7 + 2 * _NBUF:7 + 3 * _NBUF])
    dsem = rest[7 + 3 * _NBUF]

    cid = lax.axis_index("c")
    sid = lax.axis_index("s")
    r0 = sid * rows_per_tile

    pltpu.sync_copy(z2_hbm.at[pl.ds(r0, rows_per_tile)],
                    acc_sh.at[pl.ds(r0, rows_per_tile)])
    if with_deg:
      pltpu.sync_copy(z1_hbm.at[pl.ds(r0, rows_per_tile)], deg_stage)
      pltpu.sync_copy(deg_stage, deg_sh.at[pl.ds(r0, rows_per_tile)])
      for i in range(64 // 16):
        ones_v[pl.ds(i * 16, 16)] = jnp.full((16,), 1.0, jnp.float32)
    pltpu.sync_copy(cnts_hbm, cnt_v)
    plsc.subcore_barrier()

    def wait_gather(b):
      pltpu.make_async_copy(
          tbl_hbm.at[win_s.at[0]], bufs[b], gsems[b]).wait()

    def wait_scatter(b):
      pltpu.make_async_copy(
          bufs[b], acc_sh.at[win_d.at[0]], ssems[b]).wait()

    def run_region(lsrc, ldst, w):
      cvec = cnt_v[pl.ds(pl.multiple_of(w * 16, 16), 16)]
      lane16 = lax.broadcasted_iota(jnp.int32, (16,), 0)
      cnt = jnp.sum(jnp.where(lane16 == cid, cvec, 0))

      def group(gi, carry):
        @pl.when(gi * (_GRP * 64) < cnt)
        def _():
          g0 = pl.multiple_of(gi * _GRP, 8)
          pltpu.sync_copy(lsrc.at[w, pl.ds(g0, _GRP)], win_s)
          pltpu.sync_copy(ldst.at[w, pl.ds(g0, _GRP)], win_d)
          for b in range(_NBUF - 1):
            pltpu.async_copy(tbl_hbm.at[win_s.at[b]], bufs[b], gsems[b])
          for c in range(_GRP):
            b = c % _NBUF
            wait_gather(b)
            pltpu.async_copy(bufs[b], acc_sh.at[win_d.at[c]], ssems[b],
                             add=True)
            if with_deg:
              pltpu.async_copy(ones_v, deg_sh.at[win_d.at[c]], dsem,
                               add=True)
            if c + _NBUF - 1 < _GRP:
              nb = (c + _NBUF - 1) % _NBUF
              if c >= 1:
                wait_scatter(nb)
              pltpu.async_copy(tbl_hbm.at[win_s.at[c + _NBUF - 1]],
                               bufs[nb], gsems[nb])
          for c in range(_GRP - _NBUF, _GRP):
            wait_scatter(c % _NBUF)
          if with_deg:
            for _ in range(_GRP):
              pltpu.make_async_copy(
                  ones_v, deg_sh.at[win_d.at[0]], dsem).wait()
        return carry

      lax.fori_loop(0, 3, group, 0)

    @pl.when(cid == 0)
    def _():
      run_region(las, lad, 2 * sid)
      run_region(las, lad, 2 * sid + 1)

    @pl.when(cid == 1)
    def _():
      run_region(lbs, lbd, 2 * sid)
      run_region(lbs, lbd, 2 * sid + 1)

    plsc.subcore_barrier()

    g0 = pl.multiple_of(cid * _QH + r0, 8)
    pltpu.sync_copy(acc_sh.at[pl.ds(r0, rows_per_tile)],
                    acc_out.at[pl.ds(g0, rows_per_tile)])
    if with_deg:
      pltpu.sync_copy(deg_sh.at[pl.ds(r0, rows_per_tile)], deg_stage)
      pltpu.sync_copy(deg_stage, deg_out.at[pl.ds(g0, rows_per_tile)])

  return pl.kernel(
      body, out_type=outs, mesh=mesh, scratch_types=scratch,
      compiler_params=pltpu.CompilerParams(needs_layout_passes=False))


def _dot_t(a, b):
  return lax.dot_general(a, b, (((1,), (1,)), ((), ())),
                         preferred_element_type=jnp.float32)


def _tc_layer1(nblk, n_pad):
  def body(p0, d0, xb, bb, wrel, wroot, bias, h_out, s1_out, s1_acc):
    i = pl.program_id(0)
    deg = jnp.maximum(d0[...], 1.0)
    m = p0[...] / deg
    h = _dot_t(m, wrel[...]) + _dot_t(xb[...], wroot[...]) + bias[...]
    h = jnp.maximum(h, 0.0)
    h_out[...] = h
    g = lax.broadcasted_iota(jnp.int32, (_G, 128), 0)
    onehot = (bb[0] == g).astype(jnp.float32)
    ps = lax.dot_general(onehot, h, (((1,), (0,)), ((), ())),
                         preferred_element_type=jnp.float32)

    @pl.when(i == 0)
    def _():
      s1_acc[...] = jnp.zeros_like(s1_acc)

    s1_acc[...] += ps

    @pl.when(i == nblk - 1)
    def _():
      s1_out[...] = s1_acc[...]

  bs2 = lambda: pl.BlockSpec((128, 128), lambda i: (i, 0))
  col = lambda: pl.BlockSpec((128, 1), lambda i: (i, 0))
  full = lambda r, c: pl.BlockSpec((r, c), lambda i: (0, 0))
  return pl.pallas_call(
      body,
      grid=(nblk,),
      in_specs=[bs2(), col(), bs2(),
                pl.BlockSpec((1, 1, 128), lambda i: (i, 0, 0)),
                full(128, 128), full(128, 128), full(1, 128)],
      out_specs=[bs2(), pl.BlockSpec((_G, 128), lambda i: (0, 0))],
      out_shape=[jax.ShapeDtypeStruct((n_pad, 128), jnp.float32),
                 jax.ShapeDtypeStruct((_G, 128), jnp.float32)],
      scratch_shapes=[pltpu.VMEM((_G, 128), jnp.float32)],
      compiler_params=pltpu.CompilerParams(
          dimension_semantics=("arbitrary",)),
  )


def _tc_layer2_head(nblk, n_pad):
  def body(q0, d0, hb, bb, wrel, wroot, bias, s1,
           wa, wb, bl1, w2, bl2, out, ps_acc, cnt_acc):
    i = pl.program_id(0)
    deg = jnp.maximum(d0[...], 1.0)
    m = q0[...] / deg
    h = _dot_t(m, wrel[...]) + _dot_t(hb[...], wroot[...]) + bias[...]
    h = jnp.maximum(h, 0.0)
    g = lax.broadcasted_iota(jnp.int32, (_G, 128), 0)
    onehot = (bb[0] == g).astype(jnp.float32)
    ps = lax.dot_general(onehot, h, (((1,), (0,)), ((), ())),
                         preferred_element_type=jnp.float32)
    cnt = jnp.sum(onehot, axis=1, keepdims=True)

    @pl.when(i == 0)
    def _():
      ps_acc[...] = jnp.zeros_like(ps_acc)
      cnt_acc[...] = jnp.zeros_like(cnt_acc)

    ps_acc[...] += ps
    cnt_acc[...] += jnp.broadcast_to(cnt, (_G, 128))

    @pl.when(i == nblk - 1)
    def _():
      c = jnp.maximum(cnt_acc[...], 1.0)
      pool1 = s1[...] / c
      pool2 = ps_acc[...] / c
      hid = jnp.maximum(_dot_t(pool1, wa[...]) + _dot_t(pool2, wb[...])
                        + bl1[...], 0.0)
      logits = _dot_t(hid, w2[...]) + bl2[...]
      valid = lax.broadcasted_iota(jnp.int32, (_G, 128), 1) < _C
      lm = jnp.where(valid, logits, -1e30)
      mx = jnp.max(lm, axis=1, keepdims=True)
      lse = jnp.log(jnp.sum(jnp.exp(lm - mx), axis=1, keepdims=True))
      out[...] = lm - mx - lse

  bs2 = lambda: pl.BlockSpec((128, 128), lambda i: (i, 0))
  col = lambda: pl.BlockSpec((128, 1), lambda i: (i, 0))
  full = lambda r, c: pl.BlockSpec((r, c), lambda i: (0, 0))
  return pl.pallas_call(
      body,
      grid=(nblk,),
      in_specs=[bs2(), col(), bs2(),
                pl.BlockSpec((1, 1, 128), lambda i: (i, 0, 0)),
                full(128, 128), full(128, 128), full(1, 128),
                full(_G, 128),
                full(128, 128), full(128, 128), full(1, 128),
                full(128, 128), full(1, 128)],
      out_specs=[pl.BlockSpec((_G, 128), lambda i: (0, 0))],
      out_shape=[jax.ShapeDtypeStruct((_G, 128), jnp.float32)],
      scratch_shapes=[pltpu.VMEM((_G, 128), jnp.float32),
                      pltpu.VMEM((_G, 128), jnp.float32)],
      compiler_params=pltpu.CompilerParams(
          dimension_semantics=("arbitrary",)),
  )


def kernel(x, edge_index, batch, W_rel1, b1, W_root1, W_rel2, b2, W_root2,
           W_lin1, b_lin1, W_lin2, b_lin2):
  n, d = x.shape
  e = edge_index.shape[1]
  n_pad = _NPAD
  nblk = n_pad // 128
  e_pad = 32 * _NSL * 128

  src = edge_index[0]
  dst = edge_index[1]
  srcp = jnp.concatenate(
      [src, jnp.zeros((e_pad - e,), jnp.int32)]).reshape(32, _NSL, 128)
  dstp = jnp.concatenate(
      [dst, jnp.full((e_pad - e,), _PADDST, jnp.int32)]
  ).reshape(32, _NSL, 128)
  cpad = jnp.full((_NSL, 128), _QH, jnp.int32)
  xp = jnp.zeros((n_pad, d), jnp.float32).at[:n].set(x)
  batchp = jnp.concatenate(
      [batch, jnp.full((n_pad - n,), _G, jnp.int32)]).reshape(nblk, 1, 128)
  z2 = jnp.zeros((n_pad, d), jnp.float32)
  z1 = jnp.zeros((n_pad,), jnp.float32)

  l0s, l0d, l1s, l1d, c01 = _sc_route(0)(srcp, dstp, cpad)
  l2s, l2d, l3s, l3d, c23 = _sc_route(2 * _QH)(srcp, dstp, cpad)
  rs = lambda a: a.reshape(32, _CAP, 64)

  qsum_deg = _sc_qsum(True)
  alo, dlo = qsum_deg(xp, rs(l0s), rs(l0d), rs(l1s), rs(l1d), c01, z2, z1)
  ahi, dhi = qsum_deg(xp, rs(l2s), rs(l2d), rs(l3s), rs(l3d), c23, z2, z1)
  acc1 = jnp.concatenate([alo, ahi], axis=0)
  d0 = jnp.concatenate([dlo, dhi]).reshape(n_pad, 1)
  h1, s1 = _tc_layer1(nblk, n_pad)(
      acc1, d0, xp, batchp, W_rel1, W_root1, b1.reshape(1, 128))

  qsum = _sc_qsum(False)
  a2lo = qsum(h1, rs(l0s), rs(l0d), rs(l1s), rs(l1d), c01, z2, z1)
  a2hi = qsum(h1, rs(l2s), rs(l2d), rs(l3s), rs(l3d), c23, z2, z1)
  if isinstance(a2lo, (tuple, list)):
    a2lo = a2lo[0]
  if isinstance(a2hi, (tuple, list)):
    a2hi = a2hi[0]
  acc2 = jnp.concatenate([a2lo, a2hi], axis=0)

  w2p = jnp.zeros((128, 128), jnp.float32).at[:_C].set(W_lin2)
  bl2p = jnp.zeros((1, 128), jnp.float32).at[0, :_C].set(b_lin2)
  (logits,) = _tc_layer2_head(nblk, n_pad)(
      acc2, d0, h1, batchp,
      W_rel2, W_root2, b2.reshape(1, 128), s1,
      W_lin1[:, :128], W_lin1[:, 128:], b_lin1.reshape(1, 128),
      w2p, bl2p)
  return logits[:, :_C]


# speedup vs baseline: 4.6326x; 4.6295x over previous
import functools

import jax
import jax.numpy as jnp
from jax import lax
from jax.experimental import pallas as pl
from jax.experimental.pallas import tpu as pltpu
from jax.experimental.pallas import tpu_sc as plsc

_D = 128
_G = 64
_C = 10
_CH = 64
_NT = 16
_NBUF = 4
_GRP = 32


def _sc_segment_sum(with_deg, n_pad, nch):
  rows_per_tile = n_pad // _NT
  mesh = plsc.VectorSubcoreMesh(
      core_axis_name="c", subcore_axis_name="s", num_cores=1)
  outs = [jax.ShapeDtypeStruct((n_pad, _D), jnp.float32)]
  if with_deg:
    outs.append(jax.ShapeDtypeStruct((n_pad,), jnp.float32))
  scratch = (
      [pltpu.VMEM((_GRP, _CH), jnp.int32),
       pltpu.VMEM((_GRP, _CH), jnp.int32)]
      + [pltpu.VMEM((_CH, _D), jnp.float32) for _ in range(_NBUF)]
      + [pltpu.VMEM((_CH,), jnp.float32)]
      + [pltpu.VMEM((rows_per_tile,), jnp.float32)]
      + [pltpu.VMEM_SHARED((n_pad, _D), jnp.float32),
         pltpu.VMEM_SHARED((n_pad,), jnp.float32)]
      + [pltpu.SemaphoreType.DMA for _ in range(2 * _NBUF + 1)]
  )

  def body(h_hbm, src_hbm, dst_hbm, z2_hbm, z1_hbm, *rest):
    if with_deg:
      acc_out, deg_out = rest[0], rest[1]
      rest = rest[2:]
    else:
      acc_out, deg_out = rest[0], None
      rest = rest[1:]
    src_w, dst_w = rest[0], rest[1]
    bufs = list(rest[2:2 + _NBUF])
    ones_v = rest[2 + _NBUF]
    deg_stage = rest[3 + _NBUF]
    acc_sh = rest[4 + _NBUF]
    deg_sh = rest[5 + _NBUF]
    gsems = list(rest[6 + _NBUF:6 + 2 * _NBUF])
    ssems = list(rest[6 + 2 * _NBUF:6 + 3 * _NBUF])
    dsem = rest[6 + 3 * _NBUF]

    sid = lax.axis_index("s")
    r0 = sid * rows_per_tile

    pltpu.sync_copy(z2_hbm.at[pl.ds(r0, rows_per_tile)],
                    acc_sh.at[pl.ds(r0, rows_per_tile)])
    if with_deg:
      pltpu.sync_copy(z1_hbm.at[pl.ds(r0, rows_per_tile)], deg_stage)
      pltpu.sync_copy(deg_stage, deg_sh.at[pl.ds(r0, rows_per_tile)])
      for i in range(_CH // 16):
        ones_v[pl.ds(i * 16, 16)] = jnp.full((16,), 1.0, jnp.float32)
    plsc.subcore_barrier()

    def wait_gather(b):
      pltpu.make_async_copy(h_hbm.at[src_w.at[0]], bufs[b], gsems[b]).wait()

    def wait_scatter(b):
      pltpu.make_async_copy(
          bufs[b], acc_sh.at[dst_w.at[0]], ssems[b]).wait()

    def group(gi, carry):
      g0 = gi * _GRP
      pltpu.sync_copy(src_hbm.at[sid, pl.ds(g0, _GRP)], src_w)
      pltpu.sync_copy(dst_hbm.at[sid, pl.ds(g0, _GRP)], dst_w)
      for b in range(_NBUF - 1):
        pltpu.async_copy(h_hbm.at[src_w.at[b]], bufs[b], gsems[b])
      for c in range(_GRP):
        b = c % _NBUF
        wait_gather(b)
        pltpu.async_copy(bufs[b], acc_sh.at[dst_w.at[c]], ssems[b],
                         add=True)
        if with_deg:
          pltpu.async_copy(ones_v, deg_sh.at[dst_w.at[c]], dsem, add=True)
        if c + _NBUF - 1 < _GRP:
          nb = (c + _NBUF - 1) % _NBUF
          if c >= 1:
            wait_scatter(nb)
          pltpu.async_copy(h_hbm.at[src_w.at[c + _NBUF - 1]], bufs[nb],
                           gsems[nb])
      for c in range(_GRP - _NBUF, _GRP):
        wait_scatter(c % _NBUF)
      if with_deg:
        for _ in range(_GRP):
          pltpu.make_async_copy(ones_v, deg_sh.at[dst_w.at[0]], dsem).wait()
      return carry

    lax.fori_loop(0, nch // _GRP, group, 0)
    plsc.subcore_barrier()

    pltpu.sync_copy(acc_sh.at[pl.ds(r0, rows_per_tile)],
                    acc_out.at[pl.ds(r0, rows_per_tile)])
    if with_deg:
      pltpu.sync_copy(deg_sh.at[pl.ds(r0, rows_per_tile)], deg_stage)
      pltpu.sync_copy(deg_stage, deg_out.at[pl.ds(r0, rows_per_tile)])

  return pl.kernel(body, out_type=outs, mesh=mesh, scratch_types=scratch)


def _dot_t(a, b):
  return lax.dot_general(a, b, (((1,), (1,)), ((), ())),
                         preferred_element_type=jnp.float32)


def _tc_layer1(nblk, n_pad):
  def body(p0, d0, xb, bb, wrel, wroot, bias, h_out, s1_out, s1_acc):
    i = pl.program_id(0)
    deg = jnp.maximum(d0[...], 1.0)
    m = p0[...] / deg
    h = _dot_t(m, wrel[...]) + _dot_t(xb[...], wroot[...]) + bias[...]
    h = jnp.maximum(h, 0.0)
    h_out[...] = h
    g = lax.broadcasted_iota(jnp.int32, (_G, 128), 0)
    onehot = (bb[0] == g).astype(jnp.float32)
    ps = lax.dot_general(onehot, h, (((1,), (0,)), ((), ())),
                         preferred_element_type=jnp.float32)

    @pl.when(i == 0)
    def _():
      s1_acc[...] = jnp.zeros_like(s1_acc)

    s1_acc[...] += ps

    @pl.when(i == nblk - 1)
    def _():
      s1_out[...] = s1_acc[...]

  bs2 = lambda: pl.BlockSpec((128, 128), lambda i: (i, 0))
  col = lambda: pl.BlockSpec((128, 1), lambda i: (i, 0))
  full = lambda r, c: pl.BlockSpec((r, c), lambda i: (0, 0))
  return pl.pallas_call(
      body,
      grid=(nblk,),
      in_specs=[bs2(), col(), bs2(),
                pl.BlockSpec((1, 1, 128), lambda i: (i, 0, 0)),
                full(128, 128), full(128, 128), full(1, 128)],
      out_specs=[bs2(), pl.BlockSpec((_G, 128), lambda i: (0, 0))],
      out_shape=[jax.ShapeDtypeStruct((n_pad, 128), jnp.float32),
                 jax.ShapeDtypeStruct((_G, 128), jnp.float32)],
      scratch_shapes=[pltpu.VMEM((_G, 128), jnp.float32)],
      compiler_params=pltpu.CompilerParams(
          dimension_semantics=("arbitrary",)),
  )


def _tc_layer2_head(nblk, n_pad):
  def body(q0, d0, hb, bb, wrel, wroot, bias, s1,
           wa, wb, bl1, w2, bl2, out, ps_acc, cnt_acc):
    i = pl.program_id(0)
    deg = jnp.maximum(d0[...], 1.0)
    m = q0[...] / deg
    h = _dot_t(m, wrel[...]) + _dot_t(hb[...], wroot[...]) + bias[...]
    h = jnp.maximum(h, 0.0)
    g = lax.broadcasted_iota(jnp.int32, (_G, 128), 0)
    onehot = (bb[0] == g).astype(jnp.float32)
    ps = lax.dot_general(onehot, h, (((1,), (0,)), ((), ())),
                         preferred_element_type=jnp.float32)
    cnt = jnp.sum(onehot, axis=1, keepdims=True)

    @pl.when(i == 0)
    def _():
      ps_acc[...] = jnp.zeros_like(ps_acc)
      cnt_acc[...] = jnp.zeros_like(cnt_acc)

    ps_acc[...] += ps
    cnt_acc[...] += jnp.broadcast_to(cnt, (_G, 128))

    @pl.when(i == nblk - 1)
    def _():
      c = jnp.maximum(cnt_acc[...], 1.0)
      pool1 = s1[...] / c
      pool2 = ps_acc[...] / c
      hid = jnp.maximum(_dot_t(pool1, wa[...]) + _dot_t(pool2, wb[...])
                        + bl1[...], 0.0)
      logits = _dot_t(hid, w2[...]) + bl2[...]
      valid = lax.broadcasted_iota(jnp.int32, (_G, 128), 1) < _C
      lm = jnp.where(valid, logits, -1e30)
      mx = jnp.max(lm, axis=1, keepdims=True)
      lse = jnp.log(jnp.sum(jnp.exp(lm - mx), axis=1, keepdims=True))
      out[...] = lm - mx - lse

  bs2 = lambda: pl.BlockSpec((128, 128), lambda i: (i, 0))
  col = lambda: pl.BlockSpec((128, 1), lambda i: (i, 0))
  full = lambda r, c: pl.BlockSpec((r, c), lambda i: (0, 0))
  return pl.pallas_call(
      body,
      grid=(nblk,),
      in_specs=[bs2(), col(), bs2(),
                pl.BlockSpec((1, 1, 128), lambda i: (i, 0, 0)),
                full(128, 128), full(128, 128), full(1, 128),
                full(_G, 128),
                full(128, 128), full(128, 128), full(1, 128),
                full(128, 128), full(1, 128)],
      out_specs=[pl.BlockSpec((_G, 128), lambda i: (0, 0))],
      out_shape=[jax.ShapeDtypeStruct((_G, 128), jnp.float32)],
      scratch_shapes=[pltpu.VMEM((_G, 128), jnp.float32),
                      pltpu.VMEM((_G, 128), jnp.float32)],
      compiler_params=pltpu.CompilerParams(
          dimension_semantics=("arbitrary",)),
  )


def kernel(x, edge_index, batch, W_rel1, b1, W_root1, W_rel2, b2, W_root2,
           W_lin1, b_lin1, W_lin2, b_lin2):
  n, d = x.shape
  e = edge_index.shape[1]
  n_pad = ((n + 1 + 127) // 128) * 128
  nblk = n_pad // 128
  nch = -(-e // (_NT * _CH))
  nch = -(-nch // _GRP) * _GRP
  e_pad = _NT * nch * _CH

  src = edge_index[0]
  dst = edge_index[1]
  srcp = jnp.concatenate(
      [src, jnp.zeros((e_pad - e,), jnp.int32)]).reshape(_NT, nch, _CH)
  dstp = jnp.concatenate(
      [dst, jnp.full((e_pad - e,), n, jnp.int32)]).reshape(_NT, nch, _CH)
  xp = jnp.zeros((n_pad, d), jnp.float32).at[:n].set(x)
  batchp = jnp.concatenate(
      [batch, jnp.full((n_pad - n,), _G, jnp.int32)]).reshape(nblk, 1, 128)
  z2 = jnp.zeros((n_pad, d), jnp.float32)
  z1 = jnp.zeros((n_pad,), jnp.float32)

  acc1, degp = _sc_segment_sum(True, n_pad, nch)(xp, srcp, dstp, z2, z1)
  d0 = degp.reshape(n_pad, 1)
  h1, s1 = _tc_layer1(nblk, n_pad)(
      acc1, d0, xp, batchp, W_rel1, W_root1, b1.reshape(1, 128))

  acc2 = _sc_segment_sum(False, n_pad, nch)(h1, srcp, dstp, z2, z1)
  if isinstance(acc2, (tuple, list)):
    acc2 = acc2[0]

  w2p = jnp.zeros((128, 128), jnp.float32).at[:_C].set(W_lin2)
  bl2p = jnp.zeros((1, 128), jnp.float32).at[0, :_C].set(b_lin2)
  (logits,) = _tc_layer2_head(nblk, n_pad)(
      acc2, d0, h1, batchp,
      W_rel2, W_root2, b2.reshape(1, 128), s1,
      W_lin1[:, :128], W_lin1[:, 128:], b_lin1.reshape(1, 128),
      w2p, bl2p)
  return logits[:, :_C]
